# baseline (speedup 1.0000x reference)
import dataclasses
import functools

import jax
import jax.numpy as jnp
from jax import lax
from jax.experimental import pallas as pl
from jax.experimental.pallas import tpu as pltpu
from jax.experimental.pallas import tpu_sc as plsc

B = 4096
S = 50
D = 32
SP = 56

NC, NS, L = 2, 16, 16
NW = NC * NS
BPW = B // NW
CH = 2
GW = CH * SP
NCHUNK = BPW // CH
IDXW = BPW * SP


def _bow_sc(qpad_hbm, qlen_hbm, table_hbm, out_hbm,
            idx_v, rows0, rows1, len_v, den_v, out_v,
            sem0, sem1):
    wid = lax.axis_index("s") * NC + lax.axis_index("c")
    base = wid * BPW

    pltpu.sync_copy(qpad_hbm.at[pl.ds(base * SP, IDXW)], idx_v)
    pltpu.sync_copy(qlen_hbm.at[pl.ds(base, BPW)], len_v)

    @pl.loop(0, BPW // L)
    def _(i):
        lv = len_v[pl.ds(i * L, L)]
        den_v[pl.ds(i * L, L)] = lv.astype(jnp.float32) + 1e-12

    def gstart(c, buf, sem):
        pltpu.make_async_copy(
            table_hbm.at[idx_v.at[pl.ds(c * GW, GW)]], buf, sem).start()

    def gwait(c, buf, sem):
        pltpu.make_async_copy(
            table_hbm.at[idx_v.at[pl.ds(c * GW, GW)]], buf, sem).wait()

    def process(c, buf):
        for r2 in range(CH):
            row = c * CH + r2

            def body(j, accs, _r2=r2):
                a0, a1 = accs
                r = _r2 * SP + j
                return (a0 + buf[r, pl.ds(0, L)], a1 + buf[r, pl.ds(L, L)])

            z = jnp.zeros((L,), jnp.float32)
            a0, a1 = lax.fori_loop(0, S, body, (z, z), unroll=5)
            d = plsc.load_gather(den_v, [jnp.full((L,), row, jnp.int32)])
            out_v[row, pl.ds(0, L)] = a0 / d
            out_v[row, pl.ds(L, L)] = a1 / d

    gstart(0, rows0, sem0)

    @pl.loop(0, NCHUNK, step=2)
    def _(g):
        gstart(g + 1, rows1, sem1)
        gwait(g, rows0, sem0)
        process(g, rows0)

        @pl.when(g + 2 < NCHUNK)
        def _():
            gstart(g + 2, rows0, sem0)

        gwait(g + 1, rows1, sem1)
        process(g + 1, rows1)

    pltpu.sync_copy(out_v, out_hbm.at[pl.ds(base, BPW)])


@jax.jit
def _bow(qpad_flat, q_len, table):
    mesh = plsc.VectorSubcoreMesh(core_axis_name="c", subcore_axis_name="s",
                                  num_cores=NC, num_subcores=NS)
    cp = pltpu.CompilerParams()
    for field, val in (("needs_layout_passes", False),
                       ("use_tc_tiling_on_sc", False)):
        if field in pltpu.CompilerParams.__dataclass_fields__:
            cp = dataclasses.replace(cp, **{field: val})
    run = pl.kernel(
        _bow_sc,
        out_type=jax.ShapeDtypeStruct((B, D), jnp.float32),
        mesh=mesh,
        scratch_types=[
            pltpu.VMEM((IDXW,), jnp.int32),
            pltpu.VMEM((GW, D), jnp.float32),
            pltpu.VMEM((GW, D), jnp.float32),
            pltpu.VMEM((BPW,), jnp.int32),
            pltpu.VMEM((BPW,), jnp.float32),
            pltpu.VMEM((BPW, D), jnp.float32),
            pltpu.SemaphoreType.DMA,
            pltpu.SemaphoreType.DMA,
        ],
        compiler_params=cp,
    )
    return run(qpad_flat, q_len, table)


def kernel(q, q_len, table):
    qpad = jnp.pad(q.astype(jnp.int32), ((0, 0), (0, SP - S))).reshape(-1)
    return _bow(qpad, q_len.astype(jnp.int32), table)
